# Initial kernel scaffold; baseline (speedup 1.0000x reference)
import jax
import jax.numpy as jnp
from jax.experimental import pallas as pl


def kernel(x, edge_index, cand_edges, W1a, b1a, W2a, b2a, W1b, b1b, W2b, b2b, Ws, bs):
    raise NotImplementedError("write your pallas kernel here")


# speedup vs baseline: 22.3869x; 22.3869x over previous
import functools

import jax
import jax.numpy as jnp
from jax import lax
from jax.experimental import pallas as pl
from jax.experimental.pallas import tpu as pltpu
from jax.experimental.pallas import tpu_sc as plsc

NN = 100000
NNP = 102400
NE = 3200000
NEP = 3211264
NEROWS = NEP // 128
NCAND = 1000000
NCP = 1048576
NCROWS = NCP // 128

_MESH = dict(core_axis_name="c", subcore_axis_name="s")
_NSLICE = NNP // 16


def _worker_id():
    return lax.axis_index("c") * 16 + lax.axis_index("s")


ROWS_PER_CHUNK_A = 16
CHUNKS_A = NEROWS // 32 // ROWS_PER_CHUNK_A


def _agg8_body(src2d, dst2d, xt, zinit, out, sbuf, dbuf, rows, acc,
               gsem, ssem):
    c = lax.axis_index("c")
    s = lax.axis_index("s")
    wid = _worker_id()
    pltpu.sync_copy(zinit.at[pl.ds(s * _NSLICE, _NSLICE)],
                    acc.at[pl.ds(s * _NSLICE, _NSLICE)])
    plsc.subcore_barrier()
    base_row = wid * (NEROWS // 32)

    def chunk(i, carry):
        rb = base_row + i * ROWS_PER_CHUNK_A
        pltpu.sync_copy(src2d.at[pl.ds(rb, ROWS_PER_CHUNK_A)], sbuf)
        pltpu.sync_copy(dst2d.at[pl.ds(rb, ROWS_PER_CHUNK_A)], dbuf)
        for j in range(ROWS_PER_CHUNK_A):
            pltpu.async_copy(xt.at[sbuf.at[j]], rows.at[j], gsem)
        for j in range(ROWS_PER_CHUNK_A):
            pltpu.make_async_copy(xt.at[sbuf.at[j]], rows.at[j], gsem).wait()
        for j in range(ROWS_PER_CHUNK_A):
            pltpu.async_copy(rows.at[j], acc.at[dbuf.at[j]], ssem, add=True)
        for j in range(ROWS_PER_CHUNK_A):
            pltpu.make_async_copy(rows.at[j], acc.at[dbuf.at[j]], ssem).wait()
        return carry

    lax.fori_loop(0, CHUNKS_A, chunk, 0)
    plsc.subcore_barrier()
    pltpu.sync_copy(acc.at[pl.ds(s * _NSLICE, _NSLICE)],
                    out.at[c].at[pl.ds(s * _NSLICE, _NSLICE)])


def _agg8(src2d, dst2d, xt, zinit):
    return pl.kernel(
        _agg8_body,
        compiler_params=pltpu.CompilerParams(use_tc_tiling_on_sc=False),
        out_type=jax.ShapeDtypeStruct((2, NNP, 8), jnp.float32),
        mesh=plsc.VectorSubcoreMesh(**_MESH),
        scratch_types=[
            pltpu.VMEM((ROWS_PER_CHUNK_A, 128), jnp.int32),
            pltpu.VMEM((ROWS_PER_CHUNK_A, 128), jnp.int32),
            pltpu.VMEM((ROWS_PER_CHUNK_A, 128, 8), jnp.float32),
            pltpu.MemorySpace.VMEM_SHARED((NNP, 8), jnp.float32),
            pltpu.SemaphoreType.DMA,
            pltpu.SemaphoreType.DMA,
        ],
    )(src2d, dst2d, xt, zinit)


ROWS_PER_CHUNK_C = 8
CHUNKS_C = NEROWS // 16 // ROWS_PER_CHUNK_C


def _agg16_body(src2d, dst2d, gt, zinit, out, sbuf, dbuf, rows, acc,
                gsem, ssem):
    c = lax.axis_index("c")
    s = lax.axis_index("s")
    pltpu.sync_copy(zinit.at[pl.ds(s * _NSLICE, _NSLICE)],
                    acc.at[pl.ds(s * _NSLICE, _NSLICE)])
    plsc.subcore_barrier()
    base_row = s * (NEROWS // 16)
    tbl = gt.at[c]

    def chunk(i, carry):
        rb = base_row + i * ROWS_PER_CHUNK_C
        pltpu.sync_copy(src2d.at[pl.ds(rb, ROWS_PER_CHUNK_C)], sbuf)
        pltpu.sync_copy(dst2d.at[pl.ds(rb, ROWS_PER_CHUNK_C)], dbuf)
        for j in range(ROWS_PER_CHUNK_C):
            pltpu.async_copy(tbl.at[sbuf.at[j]], rows.at[j], gsem)
        for j in range(ROWS_PER_CHUNK_C):
            pltpu.make_async_copy(tbl.at[sbuf.at[j]], rows.at[j], gsem).wait()
        for j in range(ROWS_PER_CHUNK_C):
            pltpu.async_copy(rows.at[j], acc.at[dbuf.at[j]], ssem, add=True)
        for j in range(ROWS_PER_CHUNK_C):
            pltpu.make_async_copy(rows.at[j], acc.at[dbuf.at[j]], ssem).wait()
        return carry

    lax.fori_loop(0, CHUNKS_C, chunk, 0)
    plsc.subcore_barrier()
    pltpu.sync_copy(acc.at[pl.ds(s * _NSLICE, _NSLICE)],
                    out.at[c].at[pl.ds(s * _NSLICE, _NSLICE)])


def _agg16(src2d, dst2d, gt, zinit):
    return pl.kernel(
        _agg16_body,
        compiler_params=pltpu.CompilerParams(use_tc_tiling_on_sc=False),
        out_type=jax.ShapeDtypeStruct((2, NNP, 16), jnp.float32),
        mesh=plsc.VectorSubcoreMesh(**_MESH),
        scratch_types=[
            pltpu.VMEM((ROWS_PER_CHUNK_C, 128), jnp.int32),
            pltpu.VMEM((ROWS_PER_CHUNK_C, 128), jnp.int32),
            pltpu.VMEM((ROWS_PER_CHUNK_C, 128, 16), jnp.float32),
            pltpu.MemorySpace.VMEM_SHARED((NNP, 16), jnp.float32),
            pltpu.SemaphoreType.DMA,
            pltpu.SemaphoreType.DMA,
        ],
    )(src2d, dst2d, gt, zinit)


ROWS_PER_CHUNK_E = 32
CHUNKS_E = NCROWS // 32 // ROWS_PER_CHUNK_E


def _score_body(u2d, v2d, su_h, sv_h, out2d, ubuf, vbuf, suv, svv, obuf,
                su_sh, sv_sh, gsem):
    s = lax.axis_index("s")
    wid = _worker_id()
    pltpu.sync_copy(su_h.at[pl.ds(s * _NSLICE, _NSLICE)],
                    su_sh.at[pl.ds(s * _NSLICE, _NSLICE)])
    pltpu.sync_copy(sv_h.at[pl.ds(s * _NSLICE, _NSLICE)],
                    sv_sh.at[pl.ds(s * _NSLICE, _NSLICE)])
    plsc.subcore_barrier()
    base_row = wid * (NCROWS // 32)

    def chunk(i, carry):
        rb = base_row + i * ROWS_PER_CHUNK_E
        pltpu.sync_copy(u2d.at[pl.ds(rb, ROWS_PER_CHUNK_E)], ubuf)
        pltpu.sync_copy(v2d.at[pl.ds(rb, ROWS_PER_CHUNK_E)], vbuf)
        for j in range(ROWS_PER_CHUNK_E):
            pltpu.async_copy(su_sh.at[ubuf.at[j]], suv.at[j], gsem)
            pltpu.async_copy(sv_sh.at[vbuf.at[j]], svv.at[j], gsem)
        for j in range(ROWS_PER_CHUNK_E):
            pltpu.make_async_copy(su_sh.at[ubuf.at[j]], suv.at[j], gsem).wait()
            pltpu.make_async_copy(sv_sh.at[vbuf.at[j]], svv.at[j], gsem).wait()

        def vrow(j, c2):
            for l in range(8):
                z = suv[j, pl.ds(l * 16, 16)] + svv[j, pl.ds(l * 16, 16)]
                obuf[j, pl.ds(l * 16, 16)] = 1.0 / (1.0 + jnp.exp(-z))
            return c2

        lax.fori_loop(0, ROWS_PER_CHUNK_E, vrow, 0)
        pltpu.sync_copy(obuf, out2d.at[pl.ds(rb, ROWS_PER_CHUNK_E)])
        return carry

    lax.fori_loop(0, CHUNKS_E, chunk, 0)


def _score(u2d, v2d, su, sv):
    return pl.kernel(
        _score_body,
        compiler_params=pltpu.CompilerParams(use_tc_tiling_on_sc=False),
        out_type=jax.ShapeDtypeStruct((NCROWS, 128), jnp.float32),
        mesh=plsc.VectorSubcoreMesh(**_MESH),
        scratch_types=[
            pltpu.VMEM((ROWS_PER_CHUNK_E, 128), jnp.int32),
            pltpu.VMEM((ROWS_PER_CHUNK_E, 128), jnp.int32),
            pltpu.VMEM((ROWS_PER_CHUNK_E, 128), jnp.float32),
            pltpu.VMEM((ROWS_PER_CHUNK_E, 128), jnp.float32),
            pltpu.VMEM((ROWS_PER_CHUNK_E, 128), jnp.float32),
            pltpu.MemorySpace.VMEM_SHARED((NNP,), jnp.float32),
            pltpu.MemorySpace.VMEM_SHARED((NNP,), jnp.float32),
            pltpu.SemaphoreType.DMA,
        ],
    )(u2d, v2d, su, sv)


ROWS_B = 1024
GRID_B = (NNP + ROWS_B - 1) // ROWS_B


def _mlp1_body(x_ref, a0_ref, a1_ref, w1_ref, b1_ref, w2_ref, b2_ref,
               wp_ref, g_ref):
    t = x_ref[...] + a0_ref[...] + a1_ref[...]
    h = jnp.dot(t, w1_ref[...], preferred_element_type=jnp.float32,
                precision="highest") + b1_ref[...]
    h = jnp.maximum(h, 0.0)
    h = jnp.dot(h, w2_ref[...], preferred_element_type=jnp.float32,
                precision="highest") + b2_ref[...]
    h = jnp.maximum(h, 0.0)
    g = jnp.dot(h, wp_ref[...], preferred_element_type=jnp.float32,
                precision="highest")
    g_ref[0] = g[:, :16]
    g_ref[1] = g[:, 16:]


def _mlp1(x_pad, a0, a1, w1t, b1, w2t, b2, wpt):
    row_spec = pl.BlockSpec((ROWS_B, 8), lambda i: (i, 0))
    full = lambda shape: pl.BlockSpec(shape, lambda i: tuple(0 for _ in shape))
    return pl.pallas_call(
        _mlp1_body,
        grid=(GRID_B,),
        in_specs=[
            row_spec, row_spec, row_spec,
            full((8, 64)), full((1, 64)), full((64, 64)), full((1, 64)),
            full((64, 32)),
        ],
        out_specs=pl.BlockSpec((2, ROWS_B, 16), lambda i: (0, i, 0)),
        out_shape=jax.ShapeDtypeStruct((2, NNP, 16), jnp.float32),
    )(x_pad, a0, a1, w1t, b1, w2t, b2, wpt)


def _mlp2_body(g_ref, a_ref, b1_ref, w2_ref, b2_ref, wu_ref, wv_ref,
               bs_ref, su_ref, sv_ref):
    z0 = g_ref[0] + a_ref[0]
    z1 = g_ref[1] + a_ref[1]
    z = jnp.concatenate([z0, z1], axis=-1) + b1_ref[...]
    z = jnp.maximum(z, 0.0)
    h2 = jnp.dot(z, w2_ref[...], preferred_element_type=jnp.float32,
                 precision="highest") + b2_ref[...]
    su = jnp.dot(h2, wu_ref[...], preferred_element_type=jnp.float32,
                 precision="highest") + bs_ref[...]
    sv = jnp.dot(h2, wv_ref[...], preferred_element_type=jnp.float32,
                 precision="highest")
    su_ref[...] = su[:, 0]
    sv_ref[...] = sv[:, 0]


def _mlp2(g, ag, b1, w2t, b2, wu, wv, bs2d):
    pair_spec = pl.BlockSpec((2, ROWS_B, 16), lambda i: (0, i, 0))
    full = lambda shape: pl.BlockSpec(shape, lambda i: tuple(0 for _ in shape))
    return pl.pallas_call(
        _mlp2_body,
        grid=(GRID_B,),
        in_specs=[
            pair_spec, pair_spec,
            full((1, 32)), full((32, 32)), full((1, 32)),
            full((32, 1)), full((32, 1)), full((1, 1)),
        ],
        out_specs=[
            pl.BlockSpec((ROWS_B,), lambda i: (i,)),
            pl.BlockSpec((ROWS_B,), lambda i: (i,)),
        ],
        out_shape=[
            jax.ShapeDtypeStruct((NNP,), jnp.float32),
            jax.ShapeDtypeStruct((NNP,), jnp.float32),
        ],
    )(g, ag, b1, w2t, b2, wu, wv, bs2d)


def kernel(x, edge_index, cand_edges, W1a, b1a, W2a, b2a, W1b, b1b, W2b, b2b,
           Ws, bs):
    src = edge_index[0].astype(jnp.int32)
    dst = edge_index[1].astype(jnp.int32)
    epad = NN + (jnp.arange(NEP - NE, dtype=jnp.int32) % 128)
    src2d = jnp.concatenate([src, epad]).reshape(NEROWS, 128)
    dst2d = jnp.concatenate([dst, epad]).reshape(NEROWS, 128)

    x_pad = jnp.zeros((NNP, 8), jnp.float32).at[:NN, :7].set(x)
    z8 = jnp.zeros((NNP, 8), jnp.float32)
    z16 = jnp.zeros((NNP, 16), jnp.float32)

    cpad = NN + (jnp.arange(NCP - NCAND, dtype=jnp.int32) % 128)
    u2d = jnp.concatenate([cand_edges[:, 0].astype(jnp.int32), cpad]
                          ).reshape(NCROWS, 128)
    v2d = jnp.concatenate([cand_edges[:, 1].astype(jnp.int32), cpad]
                          ).reshape(NCROWS, 128)

    w1t = jnp.zeros((8, 64), jnp.float32).at[:7, :].set(W1a.T)
    b1 = b1a.reshape(1, 64)
    w2t = W2a.T
    b2 = b2a.reshape(1, 64)
    wpt = W1b.T
    b1b2 = b1b.reshape(1, 32)
    w2bt = W2b.T
    b2b2 = b2b.reshape(1, 32)
    wu = Ws[0, :32].reshape(32, 1)
    wv = Ws[0, 32:].reshape(32, 1)
    bs2d = bs.reshape(1, 1)

    agg1 = _agg8(src2d, dst2d, x_pad, z8)
    g = _mlp1(x_pad, agg1[0], agg1[1], w1t, b1, w2t, b2, wpt)
    agg2 = _agg16(src2d, dst2d, g, z16)
    su, sv = _mlp2(g, agg2, b1b2, w2bt, b2b2, wu, wv, bs2d)
    out2d = _score(u2d, v2d, su, sv)
    return out2d.reshape(-1)[:NCAND]


# speedup vs baseline: 27.0804x; 1.2097x over previous
import functools

import jax
import jax.numpy as jnp
from jax import lax
from jax.experimental import pallas as pl
from jax.experimental.pallas import tpu as pltpu
from jax.experimental.pallas import tpu_sc as plsc

NN = 100000
NNP = 102400
NE = 3200000
NEP = 3211264
NEROWS = NEP // 128
NCAND = 1000000
NCP = 1048576
NCROWS = NCP // 128

_MESH = dict(core_axis_name="c", subcore_axis_name="s")
_NSLICE = NNP // 16


def _worker_id():
    return lax.axis_index("c") * 16 + lax.axis_index("s")


ROWS_PER_CHUNK_A = 16
CHUNKS_A = NEROWS // 32 // ROWS_PER_CHUNK_A
ROWS_PER_CHUNK_C = 4
CHUNKS_C = NEROWS // 16 // ROWS_PER_CHUNK_C


def _agg_pipeline(src2d, dst2d, tbl, acc, sbuf, dbuf, rows, isem, gsem, ssem,
                  base_row, K, nchunks):
    def fetch_idx(ci, slot3):
        rb = base_row + ci * K
        pltpu.async_copy(src2d.at[pl.ds(rb, K)], sbuf.at[slot3], isem)
        pltpu.async_copy(dst2d.at[pl.ds(rb, K)], dbuf.at[slot3], isem)

    fetch_idx(0, 0)

    def chunk(i, carry):
        slot3 = lax.rem(i, 3)
        slot2 = lax.rem(i, 2)

        @pl.when(i >= 2)
        def _drain_scatter():
            for j in range(K):
                pltpu.make_async_copy(rows.at[slot2].at[j],
                                      acc.at[dbuf.at[slot3].at[j]], ssem).wait()

        pltpu.make_async_copy(src2d.at[pl.ds(0, K)], sbuf.at[slot3], isem).wait()
        pltpu.make_async_copy(dst2d.at[pl.ds(0, K)], dbuf.at[slot3], isem).wait()
        for j in range(K):
            pltpu.async_copy(tbl.at[sbuf.at[slot3].at[j]],
                             rows.at[slot2].at[j], gsem)

        @pl.when(i + 1 < nchunks)
        def _prefetch():
            fetch_idx(i + 1, lax.rem(i + 1, 3))

        for j in range(K):
            pltpu.make_async_copy(tbl.at[sbuf.at[slot3].at[j]],
                                  rows.at[slot2].at[j], gsem).wait()
        for j in range(K):
            pltpu.async_copy(rows.at[slot2].at[j],
                             acc.at[dbuf.at[slot3].at[j]], ssem, add=True)
        return carry

    lax.fori_loop(0, nchunks, chunk, 0)
    for j in range(2 * K):
        pltpu.make_async_copy(rows.at[0].at[0],
                              acc.at[dbuf.at[0].at[0]], ssem).wait()


def _agg8_body(src2d, dst2d, xt, zinit, out, sbuf, dbuf, rows, acc,
               isem, gsem, ssem):
    c = lax.axis_index("c")
    s = lax.axis_index("s")
    wid = _worker_id()
    pltpu.sync_copy(zinit.at[pl.ds(s * _NSLICE, _NSLICE)],
                    acc.at[pl.ds(s * _NSLICE, _NSLICE)])
    plsc.subcore_barrier()
    _agg_pipeline(src2d, dst2d, xt, acc, sbuf, dbuf, rows, isem, gsem, ssem,
                  wid * (NEROWS // 32), ROWS_PER_CHUNK_A, CHUNKS_A)
    plsc.subcore_barrier()
    pltpu.sync_copy(acc.at[pl.ds(s * _NSLICE, _NSLICE)],
                    out.at[c].at[pl.ds(s * _NSLICE, _NSLICE)])


def _agg8(src2d, dst2d, xt, zinit):
    return pl.kernel(
        _agg8_body,
        compiler_params=pltpu.CompilerParams(use_tc_tiling_on_sc=False),
        out_type=jax.ShapeDtypeStruct((2, NNP, 8), jnp.float32),
        mesh=plsc.VectorSubcoreMesh(**_MESH),
        scratch_types=[
            pltpu.VMEM((3, ROWS_PER_CHUNK_A, 128), jnp.int32),
            pltpu.VMEM((3, ROWS_PER_CHUNK_A, 128), jnp.int32),
            pltpu.VMEM((2, ROWS_PER_CHUNK_A, 128, 8), jnp.float32),
            pltpu.MemorySpace.VMEM_SHARED((NNP, 8), jnp.float32),
            pltpu.SemaphoreType.DMA,
            pltpu.SemaphoreType.DMA,
            pltpu.SemaphoreType.DMA,
        ],
    )(src2d, dst2d, xt, zinit)


def _agg16_body(src2d, dst2d, gt, zinit, out, sbuf, dbuf, rows, acc,
                isem, gsem, ssem):
    c = lax.axis_index("c")
    s = lax.axis_index("s")
    pltpu.sync_copy(zinit.at[pl.ds(s * _NSLICE, _NSLICE)],
                    acc.at[pl.ds(s * _NSLICE, _NSLICE)])
    plsc.subcore_barrier()
    _agg_pipeline(src2d, dst2d, gt.at[c], acc, sbuf, dbuf, rows,
                  isem, gsem, ssem,
                  s * (NEROWS // 16), ROWS_PER_CHUNK_C, CHUNKS_C)
    plsc.subcore_barrier()
    pltpu.sync_copy(acc.at[pl.ds(s * _NSLICE, _NSLICE)],
                    out.at[c].at[pl.ds(s * _NSLICE, _NSLICE)])


def _agg16(src2d, dst2d, gt, zinit):
    return pl.kernel(
        _agg16_body,
        compiler_params=pltpu.CompilerParams(use_tc_tiling_on_sc=False),
        out_type=jax.ShapeDtypeStruct((2, NNP, 16), jnp.float32),
        mesh=plsc.VectorSubcoreMesh(**_MESH),
        scratch_types=[
            pltpu.VMEM((3, ROWS_PER_CHUNK_C, 128), jnp.int32),
            pltpu.VMEM((3, ROWS_PER_CHUNK_C, 128), jnp.int32),
            pltpu.VMEM((2, ROWS_PER_CHUNK_C, 128, 16), jnp.float32),
            pltpu.MemorySpace.VMEM_SHARED((NNP, 16), jnp.float32),
            pltpu.SemaphoreType.DMA,
            pltpu.SemaphoreType.DMA,
            pltpu.SemaphoreType.DMA,
        ],
    )(src2d, dst2d, gt, zinit)


ROWS_PER_CHUNK_E = 32
CHUNKS_E = NCROWS // 32 // ROWS_PER_CHUNK_E


def _score_body(u2d, v2d, su_h, sv_h, out2d, ubuf, vbuf, suv, svv, obuf,
                su_sh, sv_sh, gsem):
    s = lax.axis_index("s")
    wid = _worker_id()
    pltpu.sync_copy(su_h.at[pl.ds(s * _NSLICE, _NSLICE)],
                    su_sh.at[pl.ds(s * _NSLICE, _NSLICE)])
    pltpu.sync_copy(sv_h.at[pl.ds(s * _NSLICE, _NSLICE)],
                    sv_sh.at[pl.ds(s * _NSLICE, _NSLICE)])
    plsc.subcore_barrier()
    base_row = wid * (NCROWS // 32)

    def chunk(i, carry):
        rb = base_row + i * ROWS_PER_CHUNK_E
        pltpu.sync_copy(u2d.at[pl.ds(rb, ROWS_PER_CHUNK_E)], ubuf)
        pltpu.sync_copy(v2d.at[pl.ds(rb, ROWS_PER_CHUNK_E)], vbuf)
        for j in range(ROWS_PER_CHUNK_E):
            pltpu.async_copy(su_sh.at[ubuf.at[j]], suv.at[j], gsem)
            pltpu.async_copy(sv_sh.at[vbuf.at[j]], svv.at[j], gsem)
        for j in range(ROWS_PER_CHUNK_E):
            pltpu.make_async_copy(su_sh.at[ubuf.at[j]], suv.at[j], gsem).wait()
            pltpu.make_async_copy(sv_sh.at[vbuf.at[j]], svv.at[j], gsem).wait()

        def vrow(j, c2):
            for l in range(8):
                z = suv[j, pl.ds(l * 16, 16)] + svv[j, pl.ds(l * 16, 16)]
                obuf[j, pl.ds(l * 16, 16)] = 1.0 / (1.0 + jnp.exp(-z))
            return c2

        lax.fori_loop(0, ROWS_PER_CHUNK_E, vrow, 0)
        pltpu.sync_copy(obuf, out2d.at[pl.ds(rb, ROWS_PER_CHUNK_E)])
        return carry

    lax.fori_loop(0, CHUNKS_E, chunk, 0)


def _score(u2d, v2d, su, sv):
    return pl.kernel(
        _score_body,
        compiler_params=pltpu.CompilerParams(use_tc_tiling_on_sc=False),
        out_type=jax.ShapeDtypeStruct((NCROWS, 128), jnp.float32),
        mesh=plsc.VectorSubcoreMesh(**_MESH),
        scratch_types=[
            pltpu.VMEM((ROWS_PER_CHUNK_E, 128), jnp.int32),
            pltpu.VMEM((ROWS_PER_CHUNK_E, 128), jnp.int32),
            pltpu.VMEM((ROWS_PER_CHUNK_E, 128), jnp.float32),
            pltpu.VMEM((ROWS_PER_CHUNK_E, 128), jnp.float32),
            pltpu.VMEM((ROWS_PER_CHUNK_E, 128), jnp.float32),
            pltpu.MemorySpace.VMEM_SHARED((NNP,), jnp.float32),
            pltpu.MemorySpace.VMEM_SHARED((NNP,), jnp.float32),
            pltpu.SemaphoreType.DMA,
        ],
    )(u2d, v2d, su, sv)


ROWS_B = 4096
GRID_B = NNP // ROWS_B


def _mlp1_body(x_ref, a0_ref, a1_ref, w1_ref, b1_ref, w2_ref, b2_ref,
               wp_ref, g_ref):
    t = x_ref[...] + a0_ref[...] + a1_ref[...]
    h = jnp.dot(t, w1_ref[...], preferred_element_type=jnp.float32,
                precision="highest") + b1_ref[...]
    h = jnp.maximum(h, 0.0)
    h = jnp.dot(h, w2_ref[...], preferred_element_type=jnp.float32,
                precision="highest") + b2_ref[...]
    h = jnp.maximum(h, 0.0)
    g = jnp.dot(h, wp_ref[...], preferred_element_type=jnp.float32,
                precision="highest")
    g_ref[0] = g[:, :16]
    g_ref[1] = g[:, 16:]


def _mlp1(x_pad, a0, a1, w1t, b1, w2t, b2, wpt):
    row_spec = pl.BlockSpec((ROWS_B, 8), lambda i: (i, 0))
    full = lambda shape: pl.BlockSpec(shape, lambda i: tuple(0 for _ in shape))
    return pl.pallas_call(
        _mlp1_body,
        grid=(GRID_B,),
        in_specs=[
            row_spec, row_spec, row_spec,
            full((8, 64)), full((1, 64)), full((64, 64)), full((1, 64)),
            full((64, 32)),
        ],
        out_specs=pl.BlockSpec((2, ROWS_B, 16), lambda i: (0, i, 0)),
        out_shape=jax.ShapeDtypeStruct((2, NNP, 16), jnp.float32),
    )(x_pad, a0, a1, w1t, b1, w2t, b2, wpt)


def _mlp2_body(g_ref, a_ref, b1_ref, w2_ref, b2_ref, wu_ref, wv_ref,
               bs_ref, su_ref, sv_ref):
    z0 = g_ref[0] + a_ref[0]
    z1 = g_ref[1] + a_ref[1]
    z = jnp.concatenate([z0, z1], axis=-1) + b1_ref[...]
    z = jnp.maximum(z, 0.0)
    h2 = jnp.dot(z, w2_ref[...], preferred_element_type=jnp.float32,
                 precision="highest") + b2_ref[...]
    su = jnp.dot(h2, wu_ref[...], preferred_element_type=jnp.float32,
                 precision="highest") + bs_ref[...]
    sv = jnp.dot(h2, wv_ref[...], preferred_element_type=jnp.float32,
                 precision="highest")
    su_ref[...] = su[:, 0]
    sv_ref[...] = sv[:, 0]


def _mlp2(g, ag, b1, w2t, b2, wu, wv, bs2d):
    pair_spec = pl.BlockSpec((2, ROWS_B, 16), lambda i: (0, i, 0))
    full = lambda shape: pl.BlockSpec(shape, lambda i: tuple(0 for _ in shape))
    return pl.pallas_call(
        _mlp2_body,
        grid=(GRID_B,),
        in_specs=[
            pair_spec, pair_spec,
            full((1, 32)), full((32, 32)), full((1, 32)),
            full((32, 1)), full((32, 1)), full((1, 1)),
        ],
        out_specs=[
            pl.BlockSpec((ROWS_B,), lambda i: (i,)),
            pl.BlockSpec((ROWS_B,), lambda i: (i,)),
        ],
        out_shape=[
            jax.ShapeDtypeStruct((NNP,), jnp.float32),
            jax.ShapeDtypeStruct((NNP,), jnp.float32),
        ],
    )(g, ag, b1, w2t, b2, wu, wv, bs2d)


def kernel(x, edge_index, cand_edges, W1a, b1a, W2a, b2a, W1b, b1b, W2b, b2b,
           Ws, bs):
    src = edge_index[0].astype(jnp.int32)
    dst = edge_index[1].astype(jnp.int32)
    epad = NN + (jnp.arange(NEP - NE, dtype=jnp.int32) % 128)
    src2d = jnp.concatenate([src, epad]).reshape(NEROWS, 128)
    dst2d = jnp.concatenate([dst, epad]).reshape(NEROWS, 128)

    x_pad = jnp.zeros((NNP, 8), jnp.float32).at[:NN, :7].set(x)
    z8 = jnp.zeros((NNP, 8), jnp.float32)
    z16 = jnp.zeros((NNP, 16), jnp.float32)

    cpad = NN + (jnp.arange(NCP - NCAND, dtype=jnp.int32) % 128)
    u2d = jnp.concatenate([cand_edges[:, 0].astype(jnp.int32), cpad]
                          ).reshape(NCROWS, 128)
    v2d = jnp.concatenate([cand_edges[:, 1].astype(jnp.int32), cpad]
                          ).reshape(NCROWS, 128)

    w1t = jnp.zeros((8, 64), jnp.float32).at[:7, :].set(W1a.T)
    b1 = b1a.reshape(1, 64)
    w2t = W2a.T
    b2 = b2a.reshape(1, 64)
    wpt = W1b.T
    b1b2 = b1b.reshape(1, 32)
    w2bt = W2b.T
    b2b2 = b2b.reshape(1, 32)
    wu = Ws[0, :32].reshape(32, 1)
    wv = Ws[0, 32:].reshape(32, 1)
    bs2d = bs.reshape(1, 1)

    agg1 = _agg8(src2d, dst2d, x_pad, z8)
    g = _mlp1(x_pad, agg1[0], agg1[1], w1t, b1, w2t, b2, wpt)
    agg2 = _agg16(src2d, dst2d, g, z16)
    su, sv = _mlp2(g, agg2, b1b2, w2bt, b2b2, wu, wv, bs2d)
    out2d = _score(u2d, v2d, su, sv)
    return out2d.reshape(-1)[:NCAND]


# speedup vs baseline: 32.1292x; 1.1864x over previous
import functools

import jax
import jax.numpy as jnp
from jax import lax
from jax.experimental import pallas as pl
from jax.experimental.pallas import tpu as pltpu
from jax.experimental.pallas import tpu_sc as plsc

NN = 100000
NNP = 100352
NE = 3200000
NEROWS = NE // 128
NCAND = 1000000
NCP = 1048576
NCROWS = NCP // 128

_MESH = dict(core_axis_name="c", subcore_axis_name="s")
_NSLICE = NNP // 16


def _worker_id():
    return lax.axis_index("c") * 16 + lax.axis_index("s")


K_A = 20
NCHUNKS_A = NEROWS // K_A
K_C = 4
NCHUNKS_C = NEROWS // K_C


def _agg_pipeline(src2d, dst2d, tbl, acc, sbuf, dbuf, rows, isem, gsem, ssem,
                  base_chunk, nchunks, K, row_slots, idx_slots):
    def fetch_idx(ci, slot):
        rb = (base_chunk + ci) * K
        pltpu.async_copy(src2d.at[pl.ds(rb, K)], sbuf.at[slot], isem)
        pltpu.async_copy(dst2d.at[pl.ds(rb, K)], dbuf.at[slot], isem)

    fetch_idx(0, 0)

    def chunk(i, carry):
        si = lax.rem(i, idx_slots)
        sr = lax.rem(i, row_slots)

        @pl.when(i >= row_slots)
        def _drain_scatter():
            for j in range(K):
                pltpu.make_async_copy(rows.at[sr].at[j],
                                      acc.at[dbuf.at[si].at[j]], ssem).wait()

        pltpu.make_async_copy(src2d.at[pl.ds(0, K)], sbuf.at[si], isem).wait()
        pltpu.make_async_copy(dst2d.at[pl.ds(0, K)], dbuf.at[si], isem).wait()
        for j in range(K):
            pltpu.async_copy(tbl.at[sbuf.at[si].at[j]],
                             rows.at[sr].at[j], gsem)

        @pl.when(i + 1 < nchunks)
        def _prefetch():
            fetch_idx(i + 1, lax.rem(i + 1, idx_slots))

        for j in range(K):
            pltpu.make_async_copy(tbl.at[sbuf.at[si].at[j]],
                                  rows.at[sr].at[j], gsem).wait()
        for j in range(K):
            pltpu.async_copy(rows.at[sr].at[j],
                             acc.at[dbuf.at[si].at[j]], ssem, add=True)
        return carry

    lax.fori_loop(0, nchunks, chunk, 0)
    for j in range(row_slots * K):
        pltpu.make_async_copy(rows.at[0].at[0],
                              acc.at[dbuf.at[0].at[0]], ssem).wait()


def _agg8_body(src2d, dst2d, xt, zinit, out, sbuf, dbuf, rows, acc,
               isem, gsem, ssem):
    c = lax.axis_index("c")
    s = lax.axis_index("s")
    wid = _worker_id()
    pltpu.sync_copy(zinit.at[pl.ds(s * _NSLICE, _NSLICE)],
                    acc.at[pl.ds(s * _NSLICE, _NSLICE)])
    plsc.subcore_barrier()
    npw = NCHUNKS_A // 32
    rem = NCHUNKS_A - npw * 32
    base_chunk = wid * npw + jnp.minimum(wid, rem)
    nchunks = npw + jnp.where(wid < rem, 1, 0)
    _agg_pipeline(src2d, dst2d, xt, acc, sbuf, dbuf, rows, isem, gsem, ssem,
                  base_chunk, nchunks, K_A, 2, 3)
    plsc.subcore_barrier()
    pltpu.sync_copy(acc.at[pl.ds(s * _NSLICE, _NSLICE)],
                    out.at[c].at[pl.ds(s * _NSLICE, _NSLICE)])


def _agg8(src2d, dst2d, xt, zinit):
    return pl.kernel(
        _agg8_body,
        compiler_params=pltpu.CompilerParams(use_tc_tiling_on_sc=False),
        out_type=jax.ShapeDtypeStruct((2, NNP, 8), jnp.float32),
        mesh=plsc.VectorSubcoreMesh(**_MESH),
        scratch_types=[
            pltpu.VMEM((3, K_A, 128), jnp.int32),
            pltpu.VMEM((3, K_A, 128), jnp.int32),
            pltpu.VMEM((2, K_A, 128, 8), jnp.float32),
            pltpu.MemorySpace.VMEM_SHARED((NNP, 8), jnp.float32),
            pltpu.SemaphoreType.DMA,
            pltpu.SemaphoreType.DMA,
            pltpu.SemaphoreType.DMA,
        ],
    )(src2d, dst2d, xt, zinit)


def _agg16_body(src2d, dst2d, gt, zinit, out, sbuf, dbuf, rows, acc,
                isem, gsem, ssem):
    c = lax.axis_index("c")
    s = lax.axis_index("s")
    pltpu.sync_copy(zinit.at[pl.ds(s * _NSLICE, _NSLICE)],
                    acc.at[pl.ds(s * _NSLICE, _NSLICE)])
    plsc.subcore_barrier()
    npw = NCHUNKS_C // 16
    rem = NCHUNKS_C - npw * 16
    base_chunk = s * npw + jnp.minimum(s, rem)
    nchunks = npw + jnp.where(s < rem, 1, 0)
    _agg_pipeline(src2d, dst2d, gt.at[c], acc, sbuf, dbuf, rows,
                  isem, gsem, ssem, base_chunk, nchunks, K_C, 3, 4)
    plsc.subcore_barrier()
    pltpu.sync_copy(acc.at[pl.ds(s * _NSLICE, _NSLICE)],
                    out.at[c].at[pl.ds(s * _NSLICE, _NSLICE)])


def _agg16(src2d, dst2d, gt, zinit):
    return pl.kernel(
        _agg16_body,
        compiler_params=pltpu.CompilerParams(use_tc_tiling_on_sc=False),
        out_type=jax.ShapeDtypeStruct((2, NNP, 16), jnp.float32),
        mesh=plsc.VectorSubcoreMesh(**_MESH),
        scratch_types=[
            pltpu.VMEM((4, K_C, 128), jnp.int32),
            pltpu.VMEM((4, K_C, 128), jnp.int32),
            pltpu.VMEM((3, K_C, 128, 16), jnp.float32),
            pltpu.MemorySpace.VMEM_SHARED((NNP, 16), jnp.float32),
            pltpu.SemaphoreType.DMA,
            pltpu.SemaphoreType.DMA,
            pltpu.SemaphoreType.DMA,
        ],
    )(src2d, dst2d, gt, zinit)


ROWS_PER_CHUNK_E = 32
CHUNKS_E = NCROWS // 32 // ROWS_PER_CHUNK_E


def _score_body(u2d, v2d, su_h, sv_h, out2d, ubuf, vbuf, suv, svv, obuf,
                su_sh, sv_sh, gsem):
    s = lax.axis_index("s")
    wid = _worker_id()
    pltpu.sync_copy(su_h.at[pl.ds(s * _NSLICE, _NSLICE)],
                    su_sh.at[pl.ds(s * _NSLICE, _NSLICE)])
    pltpu.sync_copy(sv_h.at[pl.ds(s * _NSLICE, _NSLICE)],
                    sv_sh.at[pl.ds(s * _NSLICE, _NSLICE)])
    plsc.subcore_barrier()
    base_row = wid * (NCROWS // 32)

    def chunk(i, carry):
        rb = base_row + i * ROWS_PER_CHUNK_E
        pltpu.sync_copy(u2d.at[pl.ds(rb, ROWS_PER_CHUNK_E)], ubuf)
        pltpu.sync_copy(v2d.at[pl.ds(rb, ROWS_PER_CHUNK_E)], vbuf)
        for j in range(ROWS_PER_CHUNK_E):
            pltpu.async_copy(su_sh.at[ubuf.at[j]], suv.at[j], gsem)
            pltpu.async_copy(sv_sh.at[vbuf.at[j]], svv.at[j], gsem)
        for j in range(ROWS_PER_CHUNK_E):
            pltpu.make_async_copy(su_sh.at[ubuf.at[j]], suv.at[j], gsem).wait()
            pltpu.make_async_copy(sv_sh.at[vbuf.at[j]], svv.at[j], gsem).wait()

        def vrow(j, c2):
            for l in range(8):
                z = suv[j, pl.ds(l * 16, 16)] + svv[j, pl.ds(l * 16, 16)]
                obuf[j, pl.ds(l * 16, 16)] = 1.0 / (1.0 + jnp.exp(-z))
            return c2

        lax.fori_loop(0, ROWS_PER_CHUNK_E, vrow, 0)
        pltpu.sync_copy(obuf, out2d.at[pl.ds(rb, ROWS_PER_CHUNK_E)])
        return carry

    lax.fori_loop(0, CHUNKS_E, chunk, 0)


def _score(u2d, v2d, su, sv):
    return pl.kernel(
        _score_body,
        compiler_params=pltpu.CompilerParams(use_tc_tiling_on_sc=False),
        out_type=jax.ShapeDtypeStruct((NCROWS, 128), jnp.float32),
        mesh=plsc.VectorSubcoreMesh(**_MESH),
        scratch_types=[
            pltpu.VMEM((ROWS_PER_CHUNK_E, 128), jnp.int32),
            pltpu.VMEM((ROWS_PER_CHUNK_E, 128), jnp.int32),
            pltpu.VMEM((ROWS_PER_CHUNK_E, 128), jnp.float32),
            pltpu.VMEM((ROWS_PER_CHUNK_E, 128), jnp.float32),
            pltpu.VMEM((ROWS_PER_CHUNK_E, 128), jnp.float32),
            pltpu.MemorySpace.VMEM_SHARED((NNP,), jnp.float32),
            pltpu.MemorySpace.VMEM_SHARED((NNP,), jnp.float32),
            pltpu.SemaphoreType.DMA,
        ],
    )(u2d, v2d, su, sv)


ROWS_B = 2048
GRID_B = NNP // ROWS_B


def _mlp1_body(x_ref, a0_ref, a1_ref, w1_ref, b1_ref, w2_ref, b2_ref,
               wp_ref, g_ref):
    t = x_ref[...] + a0_ref[...] + a1_ref[...]
    h = jnp.dot(t, w1_ref[...], preferred_element_type=jnp.float32) + b1_ref[...]
    h = jnp.maximum(h, 0.0)
    h = jnp.dot(h, w2_ref[...], preferred_element_type=jnp.float32) + b2_ref[...]
    h = jnp.maximum(h, 0.0)
    g = jnp.dot(h, wp_ref[...], preferred_element_type=jnp.float32)
    g_ref[0] = g[:, :16]
    g_ref[1] = g[:, 16:]


def _mlp1(x_pad, a0, a1, w1t, b1, w2t, b2, wpt):
    row_spec = pl.BlockSpec((ROWS_B, 8), lambda i: (i, 0))
    full = lambda shape: pl.BlockSpec(shape, lambda i: tuple(0 for _ in shape))
    return pl.pallas_call(
        _mlp1_body,
        grid=(GRID_B,),
        in_specs=[
            row_spec, row_spec, row_spec,
            full((8, 64)), full((1, 64)), full((64, 64)), full((1, 64)),
            full((64, 32)),
        ],
        out_specs=pl.BlockSpec((2, ROWS_B, 16), lambda i: (0, i, 0)),
        out_shape=jax.ShapeDtypeStruct((2, NNP, 16), jnp.float32),
    )(x_pad, a0, a1, w1t, b1, w2t, b2, wpt)


def _mlp2_body(g_ref, a_ref, b1_ref, w2_ref, b2_ref, wu_ref, wv_ref,
               bs_ref, su_ref, sv_ref):
    z0 = g_ref[0] + a_ref[0]
    z1 = g_ref[1] + a_ref[1]
    z = jnp.concatenate([z0, z1], axis=-1) + b1_ref[...]
    z = jnp.maximum(z, 0.0)
    h2 = jnp.dot(z, w2_ref[...], preferred_element_type=jnp.float32) + b2_ref[...]
    su = jnp.dot(h2, wu_ref[...], preferred_element_type=jnp.float32) + bs_ref[...]
    sv = jnp.dot(h2, wv_ref[...], preferred_element_type=jnp.float32)
    su_ref[...] = su[:, 0]
    sv_ref[...] = sv[:, 0]


def _mlp2(g, ag, b1, w2t, b2, wu, wv, bs2d):
    pair_spec = pl.BlockSpec((2, ROWS_B, 16), lambda i: (0, i, 0))
    full = lambda shape: pl.BlockSpec(shape, lambda i: tuple(0 for _ in shape))
    return pl.pallas_call(
        _mlp2_body,
        grid=(GRID_B,),
        in_specs=[
            pair_spec, pair_spec,
            full((1, 32)), full((32, 32)), full((1, 32)),
            full((32, 1)), full((32, 1)), full((1, 1)),
        ],
        out_specs=[
            pl.BlockSpec((ROWS_B,), lambda i: (i,)),
            pl.BlockSpec((ROWS_B,), lambda i: (i,)),
        ],
        out_shape=[
            jax.ShapeDtypeStruct((NNP,), jnp.float32),
            jax.ShapeDtypeStruct((NNP,), jnp.float32),
        ],
    )(g, ag, b1, w2t, b2, wu, wv, bs2d)


def kernel(x, edge_index, cand_edges, W1a, b1a, W2a, b2a, W1b, b1b, W2b, b2b,
           Ws, bs):
    src2d = edge_index[0].astype(jnp.int32).reshape(NEROWS, 128)
    dst2d = edge_index[1].astype(jnp.int32).reshape(NEROWS, 128)

    x_pad = jnp.zeros((NNP, 8), jnp.float32).at[:NN, :7].set(x)
    z8 = jnp.zeros((NNP, 8), jnp.float32)
    z16 = jnp.zeros((NNP, 16), jnp.float32)

    cpad = NN + (jnp.arange(NCP - NCAND, dtype=jnp.int32) % 128)
    u2d = jnp.concatenate([cand_edges[:, 0].astype(jnp.int32), cpad]
                          ).reshape(NCROWS, 128)
    v2d = jnp.concatenate([cand_edges[:, 1].astype(jnp.int32), cpad]
                          ).reshape(NCROWS, 128)

    w1t = jnp.zeros((8, 64), jnp.float32).at[:7, :].set(W1a.T)
    b1 = b1a.reshape(1, 64)
    w2t = W2a.T
    b2 = b2a.reshape(1, 64)
    wpt = W1b.T
    b1b2 = b1b.reshape(1, 32)
    w2bt = W2b.T
    b2b2 = b2b.reshape(1, 32)
    wu = Ws[0, :32].reshape(32, 1)
    wv = Ws[0, 32:].reshape(32, 1)
    bs2d = bs.reshape(1, 1)

    agg1 = _agg8(src2d, dst2d, x_pad, z8)
    g = _mlp1(x_pad, agg1[0], agg1[1], w1t, b1, w2t, b2, wpt)
    agg2 = _agg16(src2d, dst2d, g, z16)
    su, sv = _mlp2(g, agg2, b1b2, w2bt, b2b2, wu, wv, bs2d)
    out2d = _score(u2d, v2d, su, sv)
    return out2d.reshape(-1)[:NCAND]
